# baseline (speedup 1.0000x reference)
import dataclasses
import functools

import jax
import jax.numpy as jnp
from jax import lax
from jax.experimental import pallas as pl
from jax.experimental.pallas import tpu as pltpu
from jax.experimental.pallas import tpu_sc as plsc

N = 10000
E = 320000
D_IN = 128
HID = 256
NC = 2
NS = 16
NW = NC * NS
SEG_EDGES = 2560
NSEGS = E // SEG_EDGES
ZROWS = 624
TAIL = N - NS * ZROWS
BLK = 1000


def _make_sc_agg(dc: int, col_split: bool, with_counts: bool,
                 chunk: int, nbuf: int, seg_edges: int):
    mesh = plsc.VectorSubcoreMesh(core_axis_name="c", subcore_axis_name="s")
    if with_counts:
        out_type = [jax.ShapeDtypeStruct((NC, N, dc), jnp.float32),
                    jax.ShapeDtypeStruct((NW * N,), jnp.float32)]
    else:
        out_type = jax.ShapeDtypeStruct((NC, N, dc), jnp.float32)
    n_tiles = NS if col_split else NW
    nseg = seg_edges // chunk
    nsegs = E // seg_edges
    n_pairs = (nsegs + n_tiles - 1) // n_tiles
    n_pairs = (n_pairs + 1) // 2
    scratch_types = [
        pltpu.VMEM((nseg, chunk), jnp.int32),
        pltpu.VMEM((nseg, chunk), jnp.int32),
        pltpu.VMEM((nseg, chunk), jnp.int32),
        pltpu.VMEM((nseg, chunk), jnp.int32),
        pltpu.VMEM((8, dc), jnp.float32),
        pltpu.VMEM_SHARED((N, dc), jnp.float32),
    ]
    scratch_types += [pltpu.VMEM((chunk, dc), jnp.float32)] * nbuf
    scratch_types += [pltpu.SemaphoreType.DMA] * (2 * nbuf + 3)
    if with_counts:
        scratch_types.append(pltpu.VMEM((N,), jnp.float32))

    def body(table_hbm, src_hbm, dst_hbm, *refs):
        if with_counts:
            out_hbm, cnt_hbm = refs[0], refs[1]
            refs = refs[2:]
            cnt_v = refs[-1]
            refs = refs[:-1]
        else:
            out_hbm = refs[0]
            refs = refs[1:]
        src_segs = (refs[0], refs[2])
        dst_segs = (refs[1], refs[3])
        zbuf, acc = refs[4], refs[5]
        rows = refs[6:6 + nbuf]
        gsems = refs[6 + nbuf:6 + 2 * nbuf]
        ssems = refs[6 + 2 * nbuf:6 + 3 * nbuf]
        zsem = refs[6 + 3 * nbuf]
        isems = refs[6 + 3 * nbuf + 1:6 + 3 * nbuf + 3]
        cid = lax.axis_index("c")
        sid = lax.axis_index("s")
        zvec = jnp.zeros((16,), jnp.float32)
        ones = jnp.ones((16,), jnp.float32)

        @pl.loop(0, 8)
        def _(r):
            @pl.loop(0, dc, step=16)
            def _(c0):
                zbuf[r, pl.ds(c0, 16)] = zvec

        zbase = sid * ZROWS

        @pl.loop(0, ZROWS, step=8)
        def _(j):
            pltpu.async_copy(zbuf, acc.at[pl.ds(zbase + j, 8)], zsem)

        @pl.loop(0, ZROWS, step=8)
        def _(j):
            pltpu.make_async_copy(zbuf, acc.at[pl.ds(zbase, 8)], zsem).wait()

        @pl.when(sid == NS - 1)
        def _():
            @pl.loop(0, TAIL, step=8)
            def _(j):
                pltpu.sync_copy(zbuf, acc.at[pl.ds(NS * ZROWS + j, 8)])

        if with_counts:
            @pl.loop(0, N, step=16)
            def _(i):
                cnt_v[pl.ds(i, 16)] = zvec

        plsc.subcore_barrier()

        t = sid if col_split else sid * NC + cid
        off = cid * N

        def hist(idx_2d, r):
            if with_counts:
                @pl.loop(0, chunk, step=16)
                def _(k):
                    plsc.addupdate_scatter(cnt_v, [idx_2d[r, pl.ds(k, 16)]],
                                           ones)

        def issue_idx(seg, par):
            ch0 = seg * nseg
            pltpu.async_copy(src_hbm.at[pl.ds(ch0, nseg)], src_segs[par],
                             isems[par])
            pltpu.async_copy(dst_hbm.at[pl.ds(ch0, nseg)], dst_segs[par],
                             isems[par])

        def wait_idx(par):
            pltpu.make_async_copy(src_hbm.at[pl.ds(0, nseg)], src_segs[par],
                                  isems[par]).wait()
            pltpu.make_async_copy(dst_hbm.at[pl.ds(0, nseg)], dst_segs[par],
                                  isems[par]).wait()

        def process_segment(par):
            src_seg, dst_seg = src_segs[par], dst_segs[par]
            if col_split:
                @pl.loop(0, nseg)
                def _(r):
                    @pl.loop(0, chunk, step=16)
                    def _(k):
                        src_seg[r, pl.ds(k, 16)] = (
                            src_seg[r, pl.ds(k, 16)] + off)

            for b in range(nbuf):
                pltpu.async_copy(table_hbm.at[src_seg.at[b]], rows[b],
                                 gsems[b])

            @pl.loop(0, nseg, step=nbuf)
            def _(j):
                for b in range(nbuf):
                    pltpu.make_async_copy(table_hbm.at[src_seg.at[0]],
                                          rows[b], gsems[b]).wait()
                    pltpu.async_copy(rows[b], acc.at[dst_seg.at[j + b]],
                                     ssems[b], add=True)
                    hist(dst_seg, j + b)
                for b in range(nbuf):
                    @pl.when(j + nbuf + b < nseg)
                    def _():
                        pltpu.make_async_copy(rows[b], acc.at[dst_seg.at[0]],
                                              ssems[b]).wait()
                        pltpu.async_copy(table_hbm.at[src_seg.at[j + nbuf + b]],
                                         rows[b], gsems[b])

            for b in range(nbuf):
                pltpu.make_async_copy(rows[b], acc.at[dst_seg.at[0]],
                                      ssems[b]).wait()

        issue_idx(t, 0)

        @pl.loop(0, n_pairs)
        def _(ip):
            for par in range(2):
                seg = t + (2 * ip + par) * n_tiles

                @pl.when(seg < nsegs)
                def _():
                    wait_idx(par)

                    @pl.when(seg + n_tiles < nsegs)
                    def _():
                        issue_idx(seg + n_tiles, 1 - par)

                    process_segment(par)

        plsc.subcore_barrier()

        pltpu.sync_copy(acc.at[pl.ds(zbase, ZROWS)],
                        out_hbm.at[cid].at[pl.ds(zbase, ZROWS)])

        @pl.when(sid == NS - 1)
        def _():
            pltpu.sync_copy(acc.at[pl.ds(NS * ZROWS, TAIL)],
                            out_hbm.at[cid].at[pl.ds(NS * ZROWS, TAIL)])

        if with_counts:
            wid = sid * NC + cid
            pltpu.sync_copy(cnt_v, cnt_hbm.at[pl.ds(wid * N, N)])

    cp = pltpu.CompilerParams()
    if "needs_layout_passes" in pltpu.CompilerParams.__dataclass_fields__:
        cp = dataclasses.replace(cp, needs_layout_passes=False)
    return pl.kernel(body, out_type=out_type, mesh=mesh,
                     scratch_types=scratch_types, compiler_params=cp)


def _tc1a_body(x_ref, w1r_ref, b1_ref, o_ref):
    o_ref[...] = (jnp.dot(x_ref[...], w1r_ref[...],
                          preferred_element_type=jnp.float32)
                  + b1_ref[...])


def _tc1b_body(s1_ref, cnt_ref, xr_ref, w1l_ref, o_ref):
    cnt = jnp.sum(cnt_ref[...], axis=1)
    r = 1.0 / jnp.maximum(cnt, 1.0)
    agg = (s1_ref[0] + s1_ref[1]) * r[:, None]
    h = (jnp.dot(agg, w1l_ref[...], preferred_element_type=jnp.float32)
         + xr_ref[...])
    h = jnp.maximum(h, 0.0)
    o_ref[0] = h[:, :HID // 2]
    o_ref[1] = h[:, HID // 2:]


def _tc2a_body(h1_ref, w2ra_ref, w2rb_ref, b2_ref, o_ref):
    o_ref[...] = (jnp.dot(h1_ref[0], w2ra_ref[...],
                          preferred_element_type=jnp.float32)
                  + jnp.dot(h1_ref[1], w2rb_ref[...],
                            preferred_element_type=jnp.float32)
                  + b2_ref[...])


def _tc2b_body(s2_ref, cnt_ref, hr2_ref, w2la_ref, w2lb_ref, wo_ref,
               bo_ref, o_ref):
    cnt = jnp.sum(cnt_ref[...], axis=1)
    r = 1.0 / jnp.maximum(cnt, 1.0)
    h = (jnp.dot(s2_ref[0] * r[:, None], w2la_ref[...],
                 preferred_element_type=jnp.float32)
         + jnp.dot(s2_ref[1] * r[:, None], w2lb_ref[...],
                   preferred_element_type=jnp.float32)
         + hr2_ref[...])
    h = jnp.maximum(h, 0.0)
    o_ref[...] = (jnp.dot(h, wo_ref[...], preferred_element_type=jnp.float32)
                  + bo_ref[...])


def _full(shape):
    return pl.BlockSpec(shape, lambda i: tuple(0 for _ in shape))


_tc1a = pl.pallas_call(
    _tc1a_body,
    grid=(N // BLK,),
    in_specs=[
        pl.BlockSpec((BLK, D_IN), lambda i: (i, 0)),
        _full((D_IN, HID)),
        _full((1, HID)),
    ],
    out_specs=pl.BlockSpec((BLK, HID), lambda i: (i, 0)),
    out_shape=jax.ShapeDtypeStruct((N, HID), jnp.float32),
)

_tc1b = pl.pallas_call(
    _tc1b_body,
    grid=(N // BLK,),
    in_specs=[
        pl.BlockSpec((NC, BLK, D_IN), lambda i: (0, i, 0)),
        pl.BlockSpec((BLK, NW), lambda i: (i, 0)),
        pl.BlockSpec((BLK, HID), lambda i: (i, 0)),
        _full((D_IN, HID)),
    ],
    out_specs=pl.BlockSpec((NC, BLK, HID // 2), lambda i: (0, i, 0)),
    out_shape=jax.ShapeDtypeStruct((NC, N, HID // 2), jnp.float32),
)

_tc2a = pl.pallas_call(
    _tc2a_body,
    grid=(N // BLK,),
    in_specs=[
        pl.BlockSpec((NC, BLK, HID // 2), lambda i: (0, i, 0)),
        _full((HID // 2, HID)),
        _full((HID // 2, HID)),
        _full((1, HID)),
    ],
    out_specs=pl.BlockSpec((BLK, HID), lambda i: (i, 0)),
    out_shape=jax.ShapeDtypeStruct((N, HID), jnp.float32),
)

_tc2b = pl.pallas_call(
    _tc2b_body,
    grid=(N // BLK,),
    in_specs=[
        pl.BlockSpec((NC, BLK, HID // 2), lambda i: (0, i, 0)),
        pl.BlockSpec((BLK, NW), lambda i: (i, 0)),
        pl.BlockSpec((BLK, HID), lambda i: (i, 0)),
        _full((HID // 2, HID)),
        _full((HID // 2, HID)),
        _full((HID, 2)),
        _full((1, 2)),
    ],
    out_specs=pl.BlockSpec((BLK, 2), lambda i: (i, 0)),
    out_shape=jax.ShapeDtypeStruct((N, 2), jnp.float32),
)

_CHUNK1, _NBUF1 = 32, 4
_CHUNK2, _NBUF2 = 32, 4
_sc_agg1 = _make_sc_agg(D_IN, col_split=False, with_counts=True,
                        chunk=_CHUNK1, nbuf=_NBUF1, seg_edges=1280)
_sc_agg2 = _make_sc_agg(HID // 2, col_split=True, with_counts=False,
                        chunk=_CHUNK2, nbuf=_NBUF2, seg_edges=1280)


def kernel(x, edge_index, W1l, b1, W1r, W2l, b2, W2r, Wo, bo):
    src = edge_index[0].astype(jnp.int32)
    dst = edge_index[1].astype(jnp.int32)

    xr = _tc1a(x, W1r, b1.reshape(1, HID))
    s1, cnt = _sc_agg1(x, src.reshape(E // _CHUNK1, _CHUNK1),
                       dst.reshape(E // _CHUNK1, _CHUNK1))
    cnt = cnt.reshape(NW, N).T
    h1 = _tc1b(s1, cnt, xr, W1l)

    h1_tab = h1.reshape(NC * N, HID // 2)
    s2 = _sc_agg2(h1_tab, src.reshape(E // _CHUNK2, _CHUNK2),
                  dst.reshape(E // _CHUNK2, _CHUNK2))
    hr2 = _tc2a(h1, W2r[:HID // 2], W2r[HID // 2:], b2.reshape(1, HID))
    out = _tc2b(s2, cnt, hr2,
                W2l[:HID // 2], W2l[HID // 2:], Wo, bo.reshape(1, 2))
    return out


# speedup vs baseline: 1.0617x; 1.0617x over previous
import dataclasses
import functools

import jax
import jax.numpy as jnp
from jax import lax
from jax.experimental import pallas as pl
from jax.experimental.pallas import tpu as pltpu
from jax.experimental.pallas import tpu_sc as plsc

N = 10000
E = 320000
D_IN = 128
HID = 256
NC = 2
NS = 16
NW = NC * NS
SEG_EDGES = 2560
NSEGS = E // SEG_EDGES
ZROWS = 624
TAIL = N - NS * ZROWS
BLK = 1000


def _make_sc_agg(dc: int, col_split: bool, with_counts: bool,
                 chunk: int, nbuf: int, seg_edges: int, dbuf: bool):
    mesh = plsc.VectorSubcoreMesh(core_axis_name="c", subcore_axis_name="s")
    if with_counts:
        out_type = [jax.ShapeDtypeStruct((NC, N, dc), jnp.float32),
                    jax.ShapeDtypeStruct((NW * N,), jnp.float32)]
    else:
        out_type = jax.ShapeDtypeStruct((NC, N, dc), jnp.float32)
    n_tiles = NS if col_split else NW
    nseg = seg_edges // chunk
    nsegs = E // seg_edges
    n_pairs = (nsegs + n_tiles - 1) // n_tiles
    n_pairs = (n_pairs + 1) // 2
    n_ibuf = 2 if dbuf else 1
    scratch_types = [pltpu.VMEM((nseg, chunk), jnp.int32)] * (2 * n_ibuf)
    scratch_types += [
        pltpu.VMEM((8, dc), jnp.float32),
        pltpu.VMEM_SHARED((N, dc), jnp.float32),
    ]
    scratch_types += [pltpu.VMEM((chunk, dc), jnp.float32)] * nbuf
    scratch_types += [pltpu.SemaphoreType.DMA] * (2 * nbuf + 3)
    if with_counts:
        scratch_types.append(pltpu.VMEM((N,), jnp.float32))

    def body(table_hbm, src_hbm, dst_hbm, *refs):
        if with_counts:
            out_hbm, cnt_hbm = refs[0], refs[1]
            refs = refs[2:]
            cnt_v = refs[-1]
            refs = refs[:-1]
        else:
            out_hbm = refs[0]
            refs = refs[1:]
        if dbuf:
            src_segs = (refs[0], refs[2])
            dst_segs = (refs[1], refs[3])
            refs = refs[4:]
        else:
            src_segs = (refs[0], refs[0])
            dst_segs = (refs[1], refs[1])
            refs = refs[2:]
        zbuf, acc = refs[0], refs[1]
        rows = refs[2:2 + nbuf]
        gsems = refs[2 + nbuf:2 + 2 * nbuf]
        ssems = refs[2 + 2 * nbuf:2 + 3 * nbuf]
        zsem = refs[2 + 3 * nbuf]
        isems = refs[2 + 3 * nbuf + 1:2 + 3 * nbuf + 3]
        cid = lax.axis_index("c")
        sid = lax.axis_index("s")
        zvec = jnp.zeros((16,), jnp.float32)
        ones = jnp.ones((16,), jnp.float32)

        @pl.loop(0, 8)
        def _(r):
            @pl.loop(0, dc, step=16)
            def _(c0):
                zbuf[r, pl.ds(c0, 16)] = zvec

        zbase = sid * ZROWS

        @pl.loop(0, ZROWS, step=8)
        def _(j):
            pltpu.async_copy(zbuf, acc.at[pl.ds(zbase + j, 8)], zsem)

        @pl.loop(0, ZROWS, step=8)
        def _(j):
            pltpu.make_async_copy(zbuf, acc.at[pl.ds(zbase, 8)], zsem).wait()

        @pl.when(sid == NS - 1)
        def _():
            @pl.loop(0, TAIL, step=8)
            def _(j):
                pltpu.sync_copy(zbuf, acc.at[pl.ds(NS * ZROWS + j, 8)])

        if with_counts:
            @pl.loop(0, N, step=16)
            def _(i):
                cnt_v[pl.ds(i, 16)] = zvec

        plsc.subcore_barrier()

        t = sid if col_split else sid * NC + cid
        off = cid * N

        def hist(idx_2d, r):
            if with_counts:
                @pl.loop(0, chunk, step=16)
                def _(k):
                    plsc.addupdate_scatter(cnt_v, [idx_2d[r, pl.ds(k, 16)]],
                                           ones)

        def issue_idx(seg, par):
            ch0 = seg * nseg
            pltpu.async_copy(src_hbm.at[pl.ds(ch0, nseg)], src_segs[par],
                             isems[par])
            pltpu.async_copy(dst_hbm.at[pl.ds(ch0, nseg)], dst_segs[par],
                             isems[par])

        def wait_idx(par):
            pltpu.make_async_copy(src_hbm.at[pl.ds(0, nseg)], src_segs[par],
                                  isems[par]).wait()
            pltpu.make_async_copy(dst_hbm.at[pl.ds(0, nseg)], dst_segs[par],
                                  isems[par]).wait()

        def process_segment(par):
            src_seg, dst_seg = src_segs[par], dst_segs[par]
            if col_split:
                @pl.loop(0, nseg)
                def _(r):
                    @pl.loop(0, chunk, step=16)
                    def _(k):
                        src_seg[r, pl.ds(k, 16)] = (
                            src_seg[r, pl.ds(k, 16)] + off)

            for b in range(nbuf):
                pltpu.async_copy(table_hbm.at[src_seg.at[b]], rows[b],
                                 gsems[b])

            @pl.loop(0, nseg, step=nbuf)
            def _(j):
                for b in range(nbuf):
                    pltpu.make_async_copy(table_hbm.at[src_seg.at[0]],
                                          rows[b], gsems[b]).wait()
                    pltpu.async_copy(rows[b], acc.at[dst_seg.at[j + b]],
                                     ssems[b], add=True)
                    hist(dst_seg, j + b)
                for b in range(nbuf):
                    @pl.when(j + nbuf + b < nseg)
                    def _():
                        pltpu.make_async_copy(rows[b], acc.at[dst_seg.at[0]],
                                              ssems[b]).wait()
                        pltpu.async_copy(table_hbm.at[src_seg.at[j + nbuf + b]],
                                         rows[b], gsems[b])

            for b in range(nbuf):
                pltpu.make_async_copy(rows[b], acc.at[dst_seg.at[0]],
                                      ssems[b]).wait()

        if dbuf:
            issue_idx(t, 0)

            @pl.loop(0, n_pairs)
            def _(ip):
                for par in range(2):
                    seg = t + (2 * ip + par) * n_tiles

                    @pl.when(seg < nsegs)
                    def _():
                        wait_idx(par)

                        @pl.when(seg + n_tiles < nsegs)
                        def _():
                            issue_idx(seg + n_tiles, 1 - par)

                        process_segment(par)
        else:
            @pl.loop(t, nsegs, step=n_tiles)
            def _(seg):
                issue_idx(seg, 0)
                wait_idx(0)
                process_segment(0)

        plsc.subcore_barrier()

        pltpu.sync_copy(acc.at[pl.ds(zbase, ZROWS)],
                        out_hbm.at[cid].at[pl.ds(zbase, ZROWS)])

        @pl.when(sid == NS - 1)
        def _():
            pltpu.sync_copy(acc.at[pl.ds(NS * ZROWS, TAIL)],
                            out_hbm.at[cid].at[pl.ds(NS * ZROWS, TAIL)])

        if with_counts:
            wid = sid * NC + cid
            pltpu.sync_copy(cnt_v, cnt_hbm.at[pl.ds(wid * N, N)])

    cp = pltpu.CompilerParams()
    if "needs_layout_passes" in pltpu.CompilerParams.__dataclass_fields__:
        cp = dataclasses.replace(cp, needs_layout_passes=False)
    return pl.kernel(body, out_type=out_type, mesh=mesh,
                     scratch_types=scratch_types, compiler_params=cp)


def _tc1a_body(x_ref, w1r_ref, b1_ref, o_ref):
    o_ref[...] = (jnp.dot(x_ref[...], w1r_ref[...],
                          preferred_element_type=jnp.float32)
                  + b1_ref[...])


def _tc1b_body(s1_ref, cnt_ref, xr_ref, w1l_ref, o_ref):
    cnt = jnp.sum(cnt_ref[...], axis=1)
    r = 1.0 / jnp.maximum(cnt, 1.0)
    agg = (s1_ref[0] + s1_ref[1]) * r[:, None]
    h = (jnp.dot(agg, w1l_ref[...], preferred_element_type=jnp.float32)
         + xr_ref[...])
    h = jnp.maximum(h, 0.0)
    o_ref[0] = h[:, :HID // 2]
    o_ref[1] = h[:, HID // 2:]


def _tc2a_body(h1_ref, w2ra_ref, w2rb_ref, b2_ref, o_ref):
    o_ref[...] = (jnp.dot(h1_ref[0], w2ra_ref[...],
                          preferred_element_type=jnp.float32)
                  + jnp.dot(h1_ref[1], w2rb_ref[...],
                            preferred_element_type=jnp.float32)
                  + b2_ref[...])


def _tc2b_body(s2_ref, cnt_ref, hr2_ref, w2la_ref, w2lb_ref, wo_ref,
               bo_ref, o_ref):
    cnt = jnp.sum(cnt_ref[...], axis=1)
    r = 1.0 / jnp.maximum(cnt, 1.0)
    h = (jnp.dot(s2_ref[0] * r[:, None], w2la_ref[...],
                 preferred_element_type=jnp.float32)
         + jnp.dot(s2_ref[1] * r[:, None], w2lb_ref[...],
                   preferred_element_type=jnp.float32)
         + hr2_ref[...])
    h = jnp.maximum(h, 0.0)
    o_ref[...] = (jnp.dot(h, wo_ref[...], preferred_element_type=jnp.float32)
                  + bo_ref[...])


def _full(shape):
    return pl.BlockSpec(shape, lambda i: tuple(0 for _ in shape))


_tc1a = pl.pallas_call(
    _tc1a_body,
    grid=(N // BLK,),
    in_specs=[
        pl.BlockSpec((BLK, D_IN), lambda i: (i, 0)),
        _full((D_IN, HID)),
        _full((1, HID)),
    ],
    out_specs=pl.BlockSpec((BLK, HID), lambda i: (i, 0)),
    out_shape=jax.ShapeDtypeStruct((N, HID), jnp.float32),
)

_tc1b = pl.pallas_call(
    _tc1b_body,
    grid=(N // BLK,),
    in_specs=[
        pl.BlockSpec((NC, BLK, D_IN), lambda i: (0, i, 0)),
        pl.BlockSpec((BLK, NW), lambda i: (i, 0)),
        pl.BlockSpec((BLK, HID), lambda i: (i, 0)),
        _full((D_IN, HID)),
    ],
    out_specs=pl.BlockSpec((NC, BLK, HID // 2), lambda i: (0, i, 0)),
    out_shape=jax.ShapeDtypeStruct((NC, N, HID // 2), jnp.float32),
)

_tc2a = pl.pallas_call(
    _tc2a_body,
    grid=(N // BLK,),
    in_specs=[
        pl.BlockSpec((NC, BLK, HID // 2), lambda i: (0, i, 0)),
        _full((HID // 2, HID)),
        _full((HID // 2, HID)),
        _full((1, HID)),
    ],
    out_specs=pl.BlockSpec((BLK, HID), lambda i: (i, 0)),
    out_shape=jax.ShapeDtypeStruct((N, HID), jnp.float32),
)

_tc2b = pl.pallas_call(
    _tc2b_body,
    grid=(N // BLK,),
    in_specs=[
        pl.BlockSpec((NC, BLK, HID // 2), lambda i: (0, i, 0)),
        pl.BlockSpec((BLK, NW), lambda i: (i, 0)),
        pl.BlockSpec((BLK, HID), lambda i: (i, 0)),
        _full((HID // 2, HID)),
        _full((HID // 2, HID)),
        _full((HID, 2)),
        _full((1, 2)),
    ],
    out_specs=pl.BlockSpec((BLK, 2), lambda i: (i, 0)),
    out_shape=jax.ShapeDtypeStruct((N, 2), jnp.float32),
)

_CHUNK1, _NBUF1 = 32, 4
_CHUNK2, _NBUF2 = 64, 4
_sc_agg1 = _make_sc_agg(D_IN, col_split=False, with_counts=True,
                        chunk=_CHUNK1, nbuf=_NBUF1, seg_edges=1280, dbuf=True)
_sc_agg2 = _make_sc_agg(HID // 2, col_split=True, with_counts=False,
                        chunk=_CHUNK2, nbuf=_NBUF2, seg_edges=2560, dbuf=False)


def kernel(x, edge_index, W1l, b1, W1r, W2l, b2, W2r, Wo, bo):
    src = edge_index[0].astype(jnp.int32)
    dst = edge_index[1].astype(jnp.int32)

    xr = _tc1a(x, W1r, b1.reshape(1, HID))
    s1, cnt = _sc_agg1(x, src.reshape(E // _CHUNK1, _CHUNK1),
                       dst.reshape(E // _CHUNK1, _CHUNK1))
    cnt = cnt.reshape(NW, N).T
    h1 = _tc1b(s1, cnt, xr, W1l)

    h1_tab = h1.reshape(NC * N, HID // 2)
    s2 = _sc_agg2(h1_tab, src.reshape(E // _CHUNK2, _CHUNK2),
                  dst.reshape(E // _CHUNK2, _CHUNK2))
    hr2 = _tc2a(h1, W2r[:HID // 2], W2r[HID // 2:], b2.reshape(1, HID))
    out = _tc2b(s2, cnt, hr2,
                W2l[:HID // 2], W2l[HID // 2:], Wo, bo.reshape(1, 2))
    return out
